# Initial kernel scaffold; baseline (speedup 1.0000x reference)
import jax
import jax.numpy as jnp
from jax.experimental import pallas as pl


def kernel(x, edge_index, W_lin, b_lin, W_l1, b_l1, W_r1, W_l2, b_l2, W_r2):
    raise NotImplementedError("write your pallas kernel here")


# speedup vs baseline: 6.9591x; 6.9591x over previous
import functools

import jax
import jax.numpy as jnp
from jax import lax
from jax.experimental import pallas as pl
from jax.experimental.pallas import tpu as pltpu
from jax.experimental.pallas import tpu_sc as plsc

N = 10000
E = 320000
D = 128
NC = 2
NS = 16
NW = NC * NS
CHUNK = 80
NCHUNKS = E // CHUNK
CPW = NCHUNKS // NW
NPAD = 10240
RPT = NPAD // NS
CW = 16
ZR = 128


def _mesh():
    return plsc.VectorSubcoreMesh(core_axis_name="c", subcore_axis_name="s",
                                  num_cores=NC, num_subcores=NS)


def _sc_sum_body(table, src3, dst3, out_p, src_v, dst_v, rows, zbuf, sem,
                 accum):
    cid = lax.axis_index("c")
    sid = lax.axis_index("s")
    wid = sid * NC + cid

    zero16 = jnp.zeros((16,), jnp.float32)

    def zrow(r, c):
        for c8 in range(D // 16):
            zbuf[r, pl.ds(c8 * 16, 16)] = zero16
        return c
    lax.fori_loop(0, ZR, zrow, 0)
    for k in range(RPT // ZR):
        pltpu.sync_copy(zbuf, accum.at[pl.ds(sid * RPT + k * ZR, ZR), :])

    plsc.subcore_barrier()

    pltpu.sync_copy(src3.at[wid], src_v)
    pltpu.sync_copy(dst3.at[wid], dst_v)

    def edge_body(j, c):
        pltpu.async_copy(table.at[src_v.at[j]], rows, sem).wait()
        pltpu.sync_copy(rows, accum.at[dst_v.at[j]], add=True)
        return c
    lax.fori_loop(0, CPW, edge_body, 0)

    plsc.subcore_barrier()

    pltpu.sync_copy(accum.at[pl.ds(sid * RPT, RPT), :],
                    out_p.at[cid, pl.ds(sid * RPT, RPT), :])


def _make_sc_sum():
    return pl.kernel(
        _sc_sum_body,
        out_type=jax.ShapeDtypeStruct((NC, NPAD, D), jnp.float32),
        mesh=_mesh(),
        scratch_types=(
            pltpu.VMEM((CPW, CHUNK), jnp.int32),
            pltpu.VMEM((CPW, CHUNK), jnp.int32),
            pltpu.VMEM((CHUNK, D), jnp.float32),
            pltpu.VMEM((ZR, D), jnp.float32),
            pltpu.SemaphoreType.DMA,
            pltpu.VMEM_SHARED((NPAD, D), jnp.float32),
        ),
        compiler_params=pltpu.CompilerParams(use_tc_tiling_on_sc=False),
        name="sc_segment_sum",
    )


def _sc_counts_body(dst3, out_c, dst_v, ones_v, zc, cnt):
    cid = lax.axis_index("c")
    sid = lax.axis_index("s")
    wid = sid * NC + cid

    zero16 = jnp.zeros((16,), jnp.float32)
    one16 = jnp.ones((16,), jnp.float32)

    def zcrow(r, c):
        zc[r, :] = zero16
        return c
    lax.fori_loop(0, RPT, zcrow, 0)
    pltpu.sync_copy(zc, cnt.at[pl.ds(sid * RPT, RPT), :])

    def orow(r, c):
        ones_v[r, :] = one16
        return c
    lax.fori_loop(0, CHUNK, orow, 0)

    plsc.subcore_barrier()

    pltpu.sync_copy(dst3.at[wid], dst_v)

    def edge_body(j, c):
        pltpu.sync_copy(ones_v, cnt.at[dst_v.at[j]], add=True)
        return c
    lax.fori_loop(0, CPW, edge_body, 0)

    plsc.subcore_barrier()

    pltpu.sync_copy(cnt.at[pl.ds(sid * RPT, RPT), :],
                    out_c.at[cid, pl.ds(sid * RPT, RPT), :])


def _make_sc_counts():
    return pl.kernel(
        _sc_counts_body,
        out_type=jax.ShapeDtypeStruct((NC, NPAD, CW), jnp.float32),
        mesh=_mesh(),
        scratch_types=(
            pltpu.VMEM((CPW, CHUNK), jnp.int32),
            pltpu.VMEM((CHUNK, CW), jnp.float32),
            pltpu.VMEM((RPT, CW), jnp.float32),
            pltpu.VMEM_SHARED((NPAD, CW), jnp.float32),
        ),
        compiler_params=pltpu.CompilerParams(use_tc_tiling_on_sc=False),
        name="sc_counts",
    )


_BN = 400


def _proj_body(x_ref, w_ref, b_ref, o_ref):
    o_ref[...] = lax.dot_general(
        x_ref[...], w_ref[...], (((1,), (1,)), ((), ())),
        preferred_element_type=jnp.float32) + b_ref[...]


def _proj(x, w, b):
    return pl.pallas_call(
        _proj_body,
        grid=(N // _BN,),
        in_specs=[
            pl.BlockSpec((_BN, D), lambda i: (i, 0)),
            pl.BlockSpec((D, D), lambda i: (0, 0)),
            pl.BlockSpec((1, D), lambda i: (0, 0)),
        ],
        out_specs=pl.BlockSpec((_BN, D), lambda i: (i, 0)),
        out_shape=jax.ShapeDtypeStruct((N, D), jnp.float32),
    )(x, w, b.reshape(1, D))


def _layer_body(use_relu, p_ref, c_ref, h_ref, wl_ref, bl_ref, wr_ref, o_ref):
    summed = p_ref[0] + p_ref[1]
    counts = c_ref[0, :, 0:1] + c_ref[1, :, 0:1]
    mean = summed / jnp.maximum(counts, 1.0)
    t = (lax.dot_general(mean, wl_ref[...], (((1,), (1,)), ((), ())),
                         preferred_element_type=jnp.float32)
         + bl_ref[...]
         + lax.dot_general(h_ref[...], wr_ref[...], (((1,), (1,)), ((), ())),
                           preferred_element_type=jnp.float32))
    if use_relu:
        t = jnp.maximum(t, 0.0)
    o_ref[...] = t + h_ref[...]


def _layer(p, c, h, wl, bl, wr, use_relu):
    return pl.pallas_call(
        functools.partial(_layer_body, use_relu),
        grid=(N // _BN,),
        in_specs=[
            pl.BlockSpec((NC, _BN, D), lambda i: (0, i, 0)),
            pl.BlockSpec((NC, _BN, CW), lambda i: (0, i, 0)),
            pl.BlockSpec((_BN, D), lambda i: (i, 0)),
            pl.BlockSpec((D, D), lambda i: (0, 0)),
            pl.BlockSpec((1, D), lambda i: (0, 0)),
            pl.BlockSpec((D, D), lambda i: (0, 0)),
        ],
        out_specs=pl.BlockSpec((_BN, D), lambda i: (i, 0)),
        out_shape=jax.ShapeDtypeStruct((N, D), jnp.float32),
    )(p, c, h, wl, bl.reshape(1, D), wr)


_sc_sum = _make_sc_sum()
_sc_counts = _make_sc_counts()


def kernel(x, edge_index, W_lin, b_lin, W_l1, b_l1, W_r1, W_l2, b_l2, W_r2):
    src3 = edge_index[0].reshape(NW, CPW, CHUNK)
    dst3 = edge_index[1].reshape(NW, CPW, CHUNK)
    h = _proj(x, W_lin, b_lin)
    c = _sc_counts(dst3)
    p = _sc_sum(h, src3, dst3)
    out1 = _layer(p, c, h, W_l1, b_l1, W_r1, True)
    q = _sc_sum(out1, src3, dst3)
    out2 = _layer(q, c, out1, W_l2, b_l2, W_r2, False)
    return (out1, out2)


# speedup vs baseline: 10.3926x; 1.4934x over previous
import functools

import jax
import jax.numpy as jnp
from jax import lax
from jax.experimental import pallas as pl
from jax.experimental.pallas import tpu as pltpu
from jax.experimental.pallas import tpu_sc as plsc

N = 10000
E = 320000
D = 128
NC = 2
NS = 16
NW = NC * NS
CHUNK = 80
NCHUNKS = E // CHUNK
CPW = NCHUNKS // NW
NPAD = 10240
RPT = NPAD // NS
CW = 16
ZR = 8


def _mesh():
    return plsc.VectorSubcoreMesh(core_axis_name="c", subcore_axis_name="s",
                                  num_cores=NC, num_subcores=NS)


def _sc_sum_body(table, src3, dst3, out_p, src_v, dst_v, *rest):
    rows = rest[0:2]
    zbuf = rest[2]
    sem = rest[3:5]
    accum = rest[5]
    cid = lax.axis_index("c")
    sid = lax.axis_index("s")
    wid = sid * NC + cid

    zero16 = jnp.zeros((16,), jnp.float32)

    def zrow(r, c):
        for c8 in range(D // 16):
            zbuf[r, pl.ds(c8 * 16, 16)] = zero16
        return c
    lax.fori_loop(0, ZR, zrow, 0)

    def zcopy(k, c):
        pltpu.sync_copy(zbuf, accum.at[pl.ds(sid * RPT + k * ZR, ZR), :])
        return c
    lax.fori_loop(0, RPT // ZR, zcopy, 0)

    plsc.subcore_barrier()

    pltpu.sync_copy(src3.at[wid], src_v)
    pltpu.sync_copy(dst3.at[wid], dst_v)

    rows_a, rows_b = rows
    sem_a, sem_b = sem

    def start_gather(j, buf, s):
        pltpu.async_copy(table.at[src_v.at[j]], buf, s)

    def wait_gather(buf, s):
        pltpu.make_async_copy(table.at[pl.ds(0, CHUNK)], buf, s).wait()

    start_gather(0, rows_a, sem_a)

    def pair_body(jp, c):
        j0 = jp * 2
        start_gather(j0 + 1, rows_b, sem_b)
        wait_gather(rows_a, sem_a)
        pltpu.sync_copy(rows_a, accum.at[dst_v.at[j0]], add=True)
        start_gather(j0 + 2, rows_a, sem_a)
        wait_gather(rows_b, sem_b)
        pltpu.sync_copy(rows_b, accum.at[dst_v.at[j0 + 1]], add=True)
        return c
    lax.fori_loop(0, (CPW - 1) // 2, pair_body, 0)

    wait_gather(rows_a, sem_a)
    pltpu.sync_copy(rows_a, accum.at[dst_v.at[CPW - 1]], add=True)

    plsc.subcore_barrier()

    pltpu.sync_copy(accum.at[pl.ds(sid * RPT, RPT), :],
                    out_p.at[cid, pl.ds(sid * RPT, RPT), :])


def _make_sc_sum():
    return pl.kernel(
        _sc_sum_body,
        out_type=jax.ShapeDtypeStruct((NC, NPAD, D), jnp.float32),
        mesh=_mesh(),
        scratch_types=(
            pltpu.VMEM((CPW, CHUNK), jnp.int32),
            pltpu.VMEM((CPW, CHUNK), jnp.int32),
            pltpu.VMEM((CHUNK, D), jnp.float32),
            pltpu.VMEM((CHUNK, D), jnp.float32),
            pltpu.VMEM((ZR, D), jnp.float32),
            pltpu.SemaphoreType.DMA,
            pltpu.SemaphoreType.DMA,
            pltpu.VMEM_SHARED((NPAD, D), jnp.float32),
        ),
        compiler_params=pltpu.CompilerParams(use_tc_tiling_on_sc=False),
        name="sc_segment_sum",
    )


def _sc_counts_body(dst3, out_c, dst_v, ones_v, zc, cnt):
    cid = lax.axis_index("c")
    sid = lax.axis_index("s")
    wid = sid * NC + cid

    zero16 = jnp.zeros((16,), jnp.float32)
    one16 = jnp.ones((16,), jnp.float32)

    def zcrow(r, c):
        zc[r, :] = zero16
        return c
    lax.fori_loop(0, RPT, zcrow, 0)
    pltpu.sync_copy(zc, cnt.at[pl.ds(sid * RPT, RPT), :])

    def orow(r, c):
        ones_v[r, :] = one16
        return c
    lax.fori_loop(0, CHUNK, orow, 0)

    plsc.subcore_barrier()

    pltpu.sync_copy(dst3.at[wid], dst_v)

    def edge_body(j, c):
        pltpu.sync_copy(ones_v, cnt.at[dst_v.at[j]], add=True)
        return c
    lax.fori_loop(0, CPW, edge_body, 0)

    plsc.subcore_barrier()

    pltpu.sync_copy(cnt.at[pl.ds(sid * RPT, RPT), :],
                    out_c.at[cid, pl.ds(sid * RPT, RPT), :])


def _make_sc_counts():
    return pl.kernel(
        _sc_counts_body,
        out_type=jax.ShapeDtypeStruct((NC, NPAD, CW), jnp.float32),
        mesh=_mesh(),
        scratch_types=(
            pltpu.VMEM((CPW, CHUNK), jnp.int32),
            pltpu.VMEM((CHUNK, CW), jnp.float32),
            pltpu.VMEM((RPT, CW), jnp.float32),
            pltpu.VMEM_SHARED((NPAD, CW), jnp.float32),
        ),
        compiler_params=pltpu.CompilerParams(use_tc_tiling_on_sc=False),
        name="sc_counts",
    )


_BN = 400


def _proj_body(x_ref, w_ref, b_ref, o_ref):
    o_ref[...] = lax.dot_general(
        x_ref[...], w_ref[...], (((1,), (1,)), ((), ())),
        preferred_element_type=jnp.float32) + b_ref[...]


def _proj(x, w, b):
    return pl.pallas_call(
        _proj_body,
        grid=(N // _BN,),
        in_specs=[
            pl.BlockSpec((_BN, D), lambda i: (i, 0)),
            pl.BlockSpec((D, D), lambda i: (0, 0)),
            pl.BlockSpec((1, D), lambda i: (0, 0)),
        ],
        out_specs=pl.BlockSpec((_BN, D), lambda i: (i, 0)),
        out_shape=jax.ShapeDtypeStruct((N, D), jnp.float32),
    )(x, w, b.reshape(1, D))


def _layer_body(use_relu, p_ref, c_ref, h_ref, wl_ref, bl_ref, wr_ref, o_ref):
    summed = p_ref[0] + p_ref[1]
    counts = c_ref[0, :, 0:1] + c_ref[1, :, 0:1]
    mean = summed / jnp.maximum(counts, 1.0)
    t = (lax.dot_general(mean, wl_ref[...], (((1,), (1,)), ((), ())),
                         preferred_element_type=jnp.float32)
         + bl_ref[...]
         + lax.dot_general(h_ref[...], wr_ref[...], (((1,), (1,)), ((), ())),
                           preferred_element_type=jnp.float32))
    if use_relu:
        t = jnp.maximum(t, 0.0)
    o_ref[...] = t + h_ref[...]


def _layer(p, c, h, wl, bl, wr, use_relu):
    return pl.pallas_call(
        functools.partial(_layer_body, use_relu),
        grid=(N // _BN,),
        in_specs=[
            pl.BlockSpec((NC, _BN, D), lambda i: (0, i, 0)),
            pl.BlockSpec((NC, _BN, CW), lambda i: (0, i, 0)),
            pl.BlockSpec((_BN, D), lambda i: (i, 0)),
            pl.BlockSpec((D, D), lambda i: (0, 0)),
            pl.BlockSpec((1, D), lambda i: (0, 0)),
            pl.BlockSpec((D, D), lambda i: (0, 0)),
        ],
        out_specs=pl.BlockSpec((_BN, D), lambda i: (i, 0)),
        out_shape=jax.ShapeDtypeStruct((N, D), jnp.float32),
    )(p, c, h, wl, bl.reshape(1, D), wr)


_sc_sum = _make_sc_sum()
_sc_counts = _make_sc_counts()


def kernel(x, edge_index, W_lin, b_lin, W_l1, b_l1, W_r1, W_l2, b_l2, W_r2):
    src3 = edge_index[0].reshape(NW, CPW, CHUNK)
    dst3 = edge_index[1].reshape(NW, CPW, CHUNK)
    h = _proj(x, W_lin, b_lin)
    c = _sc_counts(dst3)
    p = _sc_sum(h, src3, dst3)
    out1 = _layer(p, c, h, W_l1, b_l1, W_r1, True)
    q = _sc_sum(out1, src3, dst3)
    out2 = _layer(q, c, out1, W_l2, b_l2, W_r2, False)
    return (out1, out2)


# speedup vs baseline: 11.8069x; 1.1361x over previous
import functools

import jax
import jax.numpy as jnp
from jax import lax
from jax.experimental import pallas as pl
from jax.experimental.pallas import tpu as pltpu
from jax.experimental.pallas import tpu_sc as plsc

N = 10000
E = 320000
D = 128
NC = 2
NS = 16
NW = NC * NS
CHUNK = 80
NCHUNKS = E // CHUNK
CPW = NCHUNKS // NW
NPAD = 10240
RPT = NPAD // NS
CW = 16
ZR = 8


def _mesh():
    return plsc.VectorSubcoreMesh(core_axis_name="c", subcore_axis_name="s",
                                  num_cores=NC, num_subcores=NS)


def _sc_sum_body(table, packed3, out_p, packed_v, srcl, dstl,
                 rows0, rows1, rows2, g0, g1, g2, s0, s1, s2, accum):
    cid = lax.axis_index("c")
    sid = lax.axis_index("s")
    wid = sid * NC + cid

    rows = (rows0, rows1, rows2)
    gsem = (g0, g1, g2)
    ssem = (s0, s1, s2)

    zero16 = jnp.zeros((16,), jnp.float32)

    def zrow(r, c):
        for c8 in range(D // 16):
            rows0[r, pl.ds(c8 * 16, 16)] = zero16
        return c
    lax.fori_loop(0, CHUNK, zrow, 0)

    def zcopy(k, c):
        pltpu.sync_copy(rows0, accum.at[pl.ds(sid * RPT + k * CHUNK, CHUNK), :])
        return c
    lax.fori_loop(0, RPT // CHUNK, zcopy, 0)

    plsc.subcore_barrier()

    pltpu.sync_copy(packed3.at[wid], packed_v)

    mask14 = jnp.full((16,), 0x3FFF, jnp.int32)

    def unpack(j, b):
        for k in range(CHUNK // 16):
            v = packed_v[j, pl.ds(k * 16, 16)]
            srcl[b, pl.ds(k * 16, 16)] = v & mask14
            dstl[b, pl.ds(k * 16, 16)] = lax.shift_right_logical(
                v, jnp.full((16,), 14, jnp.int32)) & mask14

    def sg(b, j):
        pltpu.async_copy(table.at[srcl.at[b]], rows[b], gsem[b])

    def wg(b):
        pltpu.make_async_copy(table.at[pl.ds(0, CHUNK)], rows[b],
                              gsem[b]).wait()

    def ss(b):
        pltpu.async_copy(rows[b], accum.at[dstl.at[b]], ssem[b], add=True)

    def ws(b):
        pltpu.make_async_copy(table.at[pl.ds(0, CHUNK)], rows[b],
                              ssem[b]).wait()

    unpack(0, 0); sg(0, 0)
    unpack(1, 1); sg(1, 1)
    wg(0); ss(0)
    unpack(2, 2); sg(2, 2)
    wg(1); ss(1)
    ws(0); unpack(3, 0); sg(0, 3)
    wg(2); ss(2)
    ws(1); unpack(4, 1); sg(1, 4)

    def tri(t, c):
        j0 = 3 * t
        wg(0); ss(0)
        ws(2); unpack(j0 + 2, 2); sg(2, j0 + 2)
        wg(1); ss(1)
        ws(0); unpack(j0 + 3, 0); sg(0, j0 + 3)
        wg(2); ss(2)
        ws(1); unpack(j0 + 4, 1); sg(1, j0 + 4)
        return c
    lax.fori_loop(1, (CPW - 5) // 3 + 1, tri, 0)

    wg(0); ss(0)
    wg(1); ws(2); ss(1)
    ws(0); ws(1)

    plsc.subcore_barrier()

    pltpu.sync_copy(accum.at[pl.ds(sid * RPT, RPT), :],
                    out_p.at[cid, pl.ds(sid * RPT, RPT), :])


def _make_sc_sum():
    return pl.kernel(
        _sc_sum_body,
        out_type=jax.ShapeDtypeStruct((NC, NPAD, D), jnp.float32),
        mesh=_mesh(),
        scratch_types=(
            pltpu.VMEM((CPW, CHUNK), jnp.int32),
            pltpu.VMEM((3, CHUNK), jnp.int32),
            pltpu.VMEM((3, CHUNK), jnp.int32),
            pltpu.VMEM((CHUNK, D), jnp.float32),
            pltpu.VMEM((CHUNK, D), jnp.float32),
            pltpu.VMEM((CHUNK, D), jnp.float32),
            pltpu.SemaphoreType.DMA,
            pltpu.SemaphoreType.DMA,
            pltpu.SemaphoreType.DMA,
            pltpu.SemaphoreType.DMA,
            pltpu.SemaphoreType.DMA,
            pltpu.SemaphoreType.DMA,
            pltpu.VMEM_SHARED((NPAD, D), jnp.float32),
        ),
        compiler_params=pltpu.CompilerParams(use_tc_tiling_on_sc=False),
        name="sc_segment_sum",
    )


def _sc_counts_body(dst3, out_c, dst_v, ones_v, zc, cnt):
    cid = lax.axis_index("c")
    sid = lax.axis_index("s")
    wid = sid * NC + cid

    zero16 = jnp.zeros((16,), jnp.float32)
    one16 = jnp.ones((16,), jnp.float32)

    def zcrow(r, c):
        zc[r, :] = zero16
        return c
    lax.fori_loop(0, RPT, zcrow, 0)
    pltpu.sync_copy(zc, cnt.at[pl.ds(sid * RPT, RPT), :])

    def orow(r, c):
        ones_v[r, :] = one16
        return c
    lax.fori_loop(0, CHUNK, orow, 0)

    plsc.subcore_barrier()

    pltpu.sync_copy(dst3.at[wid], dst_v)

    def edge_body(j, c):
        pltpu.sync_copy(ones_v, cnt.at[dst_v.at[j]], add=True)
        return c
    lax.fori_loop(0, CPW, edge_body, 0)

    plsc.subcore_barrier()

    pltpu.sync_copy(cnt.at[pl.ds(sid * RPT, RPT), :],
                    out_c.at[cid, pl.ds(sid * RPT, RPT), :])


def _make_sc_counts():
    return pl.kernel(
        _sc_counts_body,
        out_type=jax.ShapeDtypeStruct((NC, NPAD, CW), jnp.float32),
        mesh=_mesh(),
        scratch_types=(
            pltpu.VMEM((CPW, CHUNK), jnp.int32),
            pltpu.VMEM((CHUNK, CW), jnp.float32),
            pltpu.VMEM((RPT, CW), jnp.float32),
            pltpu.VMEM_SHARED((NPAD, CW), jnp.float32),
        ),
        compiler_params=pltpu.CompilerParams(use_tc_tiling_on_sc=False),
        name="sc_counts",
    )


_BN = 400


def _proj_body(x_ref, w_ref, b_ref, o_ref):
    o_ref[...] = lax.dot_general(
        x_ref[...], w_ref[...], (((1,), (1,)), ((), ())),
        preferred_element_type=jnp.float32) + b_ref[...]


def _proj(x, w, b):
    return pl.pallas_call(
        _proj_body,
        grid=(N // _BN,),
        in_specs=[
            pl.BlockSpec((_BN, D), lambda i: (i, 0)),
            pl.BlockSpec((D, D), lambda i: (0, 0)),
            pl.BlockSpec((1, D), lambda i: (0, 0)),
        ],
        out_specs=pl.BlockSpec((_BN, D), lambda i: (i, 0)),
        out_shape=jax.ShapeDtypeStruct((N, D), jnp.float32),
    )(x, w, b.reshape(1, D))


def _layer_body(use_relu, p_ref, c_ref, h_ref, wl_ref, bl_ref, wr_ref, o_ref):
    summed = p_ref[0] + p_ref[1]
    counts = c_ref[0, :, 0:1] + c_ref[1, :, 0:1]
    mean = summed / jnp.maximum(counts, 1.0)
    t = (lax.dot_general(mean, wl_ref[...], (((1,), (1,)), ((), ())),
                         preferred_element_type=jnp.float32)
         + bl_ref[...]
         + lax.dot_general(h_ref[...], wr_ref[...], (((1,), (1,)), ((), ())),
                           preferred_element_type=jnp.float32))
    if use_relu:
        t = jnp.maximum(t, 0.0)
    o_ref[...] = t + h_ref[...]


def _layer(p, c, h, wl, bl, wr, use_relu):
    return pl.pallas_call(
        functools.partial(_layer_body, use_relu),
        grid=(N // _BN,),
        in_specs=[
            pl.BlockSpec((NC, _BN, D), lambda i: (0, i, 0)),
            pl.BlockSpec((NC, _BN, CW), lambda i: (0, i, 0)),
            pl.BlockSpec((_BN, D), lambda i: (i, 0)),
            pl.BlockSpec((D, D), lambda i: (0, 0)),
            pl.BlockSpec((1, D), lambda i: (0, 0)),
            pl.BlockSpec((D, D), lambda i: (0, 0)),
        ],
        out_specs=pl.BlockSpec((_BN, D), lambda i: (i, 0)),
        out_shape=jax.ShapeDtypeStruct((N, D), jnp.float32),
    )(p, c, h, wl, bl.reshape(1, D), wr)


_sc_sum = _make_sc_sum()
_sc_counts = _make_sc_counts()


def kernel(x, edge_index, W_lin, b_lin, W_l1, b_l1, W_r1, W_l2, b_l2, W_r2):
    src = edge_index[0]
    dst = edge_index[1]
    packed3 = (src | (dst << 14)).reshape(NW, CPW, CHUNK)
    dst3 = dst.reshape(NW, CPW, CHUNK)
    h = _proj(x, W_lin, b_lin)
    c = _sc_counts(dst3)
    p = _sc_sum(h, packed3)
    out1 = _layer(p, c, h, W_l1, b_l1, W_r1, True)
    q = _sc_sum(out1, packed3)
    out2 = _layer(q, c, out1, W_l2, b_l2, W_r2, False)
    return (out1, out2)
